# baseline (speedup 1.0000x reference)
import functools

import jax
import jax.numpy as jnp
from jax import lax
from jax.experimental import pallas as pl
from jax.experimental.pallas import tpu as pltpu
from jax.experimental.pallas import tpu_sc as plsc

EMB = 32
EPS = 1e-12
HALF = 16
NW = 32
CHUNK = 128
PITCH = 33


def kernel(entity_ids, table, gamma, beta):
    bsz, hist = entity_ids.shape
    nrows = bsz * hist
    rows_pw = nrows // NW
    nchunks = rows_pw // CHUNK
    bhi = bsz // CHUNK
    ids_flat = entity_ids.astype(jnp.int32).T.reshape(nrows)

    mesh = plsc.VectorSubcoreMesh(core_axis_name="c", subcore_axis_name="s")

    @functools.partial(
        pl.kernel,
        out_type=jax.ShapeDtypeStruct((hist, 4, bhi, 8 * CHUNK), jnp.float32),
        mesh=mesh,
        scratch_types=[
            pltpu.VMEM((rows_pw,), jnp.int32),
            pltpu.VMEM((2, CHUNK, EMB), jnp.float32),
            pltpu.VMEM((CHUNK * PITCH,), jnp.float32),
            pltpu.VMEM((2, 4, 8 * CHUNK), jnp.float32),
            pltpu.VMEM((EMB,), jnp.float32),
            pltpu.VMEM((EMB,), jnp.float32),
            pltpu.VMEM((EMB * HALF,), jnp.float32),
            pltpu.VMEM((EMB * HALF,), jnp.float32),
            pltpu.SemaphoreType.DMA,
            pltpu.SemaphoreType.DMA,
            pltpu.SemaphoreType.DMA,
            pltpu.SemaphoreType.DMA,
        ],
        compiler_params=pltpu.CompilerParams(
            needs_layout_passes=False, use_tc_tiling_on_sc=False),
    )
    def sc_kernel(ids_hbm, table_hbm, gamma_hbm, beta_hbm, out_hbm,
                  idx_v, data_v, pad_v, norm_v, gam_v, bet_v, gsp_v, bsp_v,
                  gsem0, gsem1, ssem0, ssem1):
        gsem = (gsem0, gsem1)
        ssem = (ssem0, ssem1)
        wid = lax.axis_index("s") * 2 + lax.axis_index("c")
        pltpu.sync_copy(ids_hbm.at[pl.ds(wid * rows_pw, rows_pw)], idx_v)
        pltpu.sync_copy(gamma_hbm, gam_v)
        pltpu.sync_copy(beta_hbm, bet_v)
        for half in range(2):
            gh = gam_v[pl.ds(half * HALF, HALF)]
            bh = bet_v[pl.ds(half * HALF, HALF)]
            for j in range(HALF):
                c = half * HALF + j
                gsp_v[pl.ds(c * HALF, HALF)] = jnp.full(
                    (HALF,), gh[j], jnp.float32)
                bsp_v[pl.ds(c * HALF, HALF)] = jnp.full(
                    (HALF,), bh[j], jnp.float32)
        iota_p = lax.iota(jnp.int32, HALF) * PITCH
        chunk0 = wid * nchunks

        def start_gather(k, slot):
            pltpu.async_copy(
                table_hbm.at[idx_v.at[pl.ds(k * CHUNK, CHUNK)]],
                data_v.at[slot], gsem[slot])

        def out_copies(k, nslot):
            g = chunk0 + k
            h = g // bhi
            b = g % bhi
            for ch in range(4):
                yield pltpu.make_async_copy(
                    norm_v.at[nslot, ch], out_hbm.at[h, ch, b], ssem[nslot])

        def process_chunk(k, slot, nslot):
            pltpu.make_async_copy(
                table_hbm.at[idx_v.at[pl.ds(k * CHUNK, CHUNK)]],
                data_v.at[slot], gsem[slot]).wait()

            @pl.when(k >= 2)
            def _():
                for cp in out_copies(k - 2, nslot):
                    cp.wait()

            def repack(r):
                pad_v[pl.ds(r * PITCH, HALF)] = \
                    data_v[slot, r, pl.ds(0, HALF)]
                pad_v[pl.ds(r * PITCH + HALF, HALF)] = \
                    data_v[slot, r, pl.ds(HALF, HALF)]

            plsc.parallel_loop(0, CHUNK, 1, unroll=8)(repack)

            @pl.when(k + 2 < nchunks)
            def _():
                start_gather(k + 2, slot)

            def group(grp):
                col = iota_p + grp * (HALF * PITCH)
                acc_s = [jnp.zeros((HALF,), jnp.float32)] * 4
                acc_q = [jnp.zeros((HALF,), jnp.float32)] * 4
                for c in range(EMB):
                    v = plsc.load_gather(pad_v, [col + c])
                    acc_s[c % 4] = acc_s[c % 4] + v
                    acc_q[c % 4] = acc_q[c % 4] + v * v
                s = (acc_s[0] + acc_s[1]) + (acc_s[2] + acc_s[3])
                q2 = (acc_q[0] + acc_q[1]) + (acc_q[2] + acc_q[3])
                mean = s * (1.0 / EMB)
                var = jnp.maximum(q2 * (1.0 / EMB) - mean * mean, 0.0) + EPS
                i = lax.bitcast_convert_type(var, jnp.int32)
                i = jnp.int32(0x5F3759DF) - lax.shift_right_logical(i, 1)
                y = lax.bitcast_convert_type(i, jnp.float32)
                xh = var * 0.5
                y = y * (1.5 - xh * y * y)
                y = y * (1.5 - xh * y * y)
                y = y * (1.5 - xh * y * y)
                for c in range(EMB):
                    v = plsc.load_gather(pad_v, [col + c])
                    gsv = gsp_v[pl.ds(c * HALF, HALF)]
                    bsv = bsp_v[pl.ds(c * HALF, HALF)]
                    o = (v - mean) * (y * gsv) + bsv
                    norm_v[nslot, c // 8,
                           pl.ds((c % 8) * CHUNK + grp * HALF, HALF)] = o

            plsc.parallel_loop(0, CHUNK // HALF, 1, unroll=2)(group)

            for cp in out_copies(k, nslot):
                cp.start()

        for k in range(2):
            start_gather(k, k)

        def pair(p, _):
            for j in range(2):
                process_chunk(2 * p + j, j, j)
            return 0

        lax.fori_loop(0, nchunks // 2, pair, 0)

        for j in range(2):
            for cp in out_copies(nchunks - 2 + j, j):
                cp.wait()

    out4 = sc_kernel(ids_flat, table, gamma, beta)
    out5 = out4.reshape(hist, 4, bhi, 8, CHUNK)
    return out5.transpose(2, 4, 0, 1, 3).reshape(bsz, hist, EMB)


# speedup vs baseline: 1.1583x; 1.1583x over previous
import functools

import jax
import jax.numpy as jnp
from jax import lax
from jax.experimental import pallas as pl
from jax.experimental.pallas import tpu as pltpu
from jax.experimental.pallas import tpu_sc as plsc

EMB = 32
EPS = 1e-12
HALF = 16
NW = 32
CHUNK = 128
UNIT_B = 1024


def kernel(entity_ids, table, gamma, beta):
    bsz, hist = entity_ids.shape
    nrows = bsz * hist
    nunits = hist * (bsz // UNIT_B)
    units_per_worker = nunits // NW
    chunks_per_unit = UNIT_B // CHUNK
    bhi_per_b = bsz // 128
    h_stride = EMB * bsz
    ids_t = entity_ids.astype(jnp.int32).T

    mesh = plsc.VectorSubcoreMesh(core_axis_name="c", subcore_axis_name="s")

    @functools.partial(
        pl.kernel,
        out_type=jax.ShapeDtypeStruct((nrows * EMB,), jnp.float32),
        mesh=mesh,
        scratch_types=[
            pltpu.VMEM((UNIT_B,), jnp.int32),
            pltpu.VMEM((2, CHUNK, EMB), jnp.float32),
            pltpu.VMEM((4 * 8 * UNIT_B,), jnp.float32),
            pltpu.VMEM((EMB,), jnp.float32),
            pltpu.VMEM((EMB,), jnp.float32),
        ] + [pltpu.SemaphoreType.DMA] * 3,
        compiler_params=pltpu.CompilerParams(
            needs_layout_passes=False, use_tc_tiling_on_sc=False),
    )
    def sc_kernel(ids_hbm, table_hbm, gamma_hbm, beta_hbm, out_hbm,
                  idx_v, data_v, stage_v, gam_v, bet_v,
                  gsem0, gsem1, ssem):
        gsem = (gsem0, gsem1)
        wid = lax.axis_index("s") * 2 + lax.axis_index("c")
        pltpu.sync_copy(gamma_hbm, gam_v)
        pltpu.sync_copy(beta_hbm, bet_v)
        g0 = gam_v[pl.ds(0, HALF)]
        g1 = gam_v[pl.ds(HALF, HALF)]
        b0 = bet_v[pl.ds(0, HALF)]
        b1 = bet_v[pl.ds(HALF, HALF)]
        lane15 = jnp.full((HALF, 1), 15, jnp.int32)
        dnums = lax.GatherDimensionNumbers(
            offset_dims=(), collapsed_slice_dims=(0,), start_index_map=(0,))

        def bcast_last(x):
            return lax.gather(x, lane15, dnums, (1,),
                              mode=lax.GatherScatterMode.PROMISE_IN_BOUNDS)

        iota = lax.iota(jnp.int32, HALF)
        k01 = (iota // 8) * (8 * UNIT_B) + (iota % 8) * 128
        k23 = k01 + 2 * (8 * UNIT_B)

        def start_gather(k, slot):
            return pltpu.async_copy(
                table_hbm.at[idx_v.at[pl.ds(k * CHUNK, CHUNK)]],
                data_v.at[slot], gsem[slot])

        def unit_body(u, _):
            h = u // (bhi_per_b // 8)
            q = u % (bhi_per_b // 8)
            pltpu.sync_copy(ids_hbm.at[h, pl.ds(q * UNIT_B, UNIT_B)], idx_v)
            start_gather(0, 0)
            for k in range(chunks_per_unit):
                slot = k % 2
                if k + 1 < chunks_per_unit:
                    start_gather(k + 1, (k + 1) % 2)
                pltpu.make_async_copy(
                    table_hbm.at[idx_v.at[pl.ds(k * CHUNK, CHUNK)]],
                    data_v.at[slot], gsem[slot]).wait()

                def row(r):
                    v0 = data_v[slot, r, pl.ds(0, HALF)]
                    v1 = data_v[slot, r, pl.ds(HALF, HALF)]
                    s = bcast_last(jnp.cumsum(v0 + v1))
                    q2 = bcast_last(jnp.cumsum(v0 * v0 + v1 * v1))
                    mean = s * (1.0 / EMB)
                    var = jnp.maximum(
                        q2 * (1.0 / EMB) - mean * mean, 0.0) + EPS
                    i = lax.bitcast_convert_type(var, jnp.int32)
                    i = (jnp.int32(0x5F3759DF)
                         - lax.shift_right_logical(i, 1))
                    y = lax.bitcast_convert_type(i, jnp.float32)
                    xh = var * 0.5
                    y = y * (1.5 - xh * y * y)
                    y = y * (1.5 - xh * y * y)
                    y = y * (1.5 - xh * y * y)
                    pos = k * 1024 + r
                    plsc.store_scatter(stage_v, [k01 + pos],
                                       (v0 - mean) * (y * g0) + b0)
                    plsc.store_scatter(stage_v, [k23 + pos],
                                       (v1 - mean) * (y * g1) + b1)

                plsc.parallel_loop(0, CHUNK, 1, unroll=8)(row)

            out_off = h * h_stride + q * (8 * UNIT_B)
            for ch in range(4):
                pltpu.async_copy(
                    stage_v.at[pl.ds(ch * 8 * UNIT_B, 8 * UNIT_B)],
                    out_hbm.at[pl.ds(out_off + ch * (8 * bsz), 8 * UNIT_B)],
                    ssem)
            for ch in range(4):
                pltpu.make_async_copy(
                    stage_v.at[pl.ds(ch * 8 * UNIT_B, 8 * UNIT_B)],
                    out_hbm.at[pl.ds(out_off + ch * (8 * bsz), 8 * UNIT_B)],
                    ssem).wait()
            return 0

        lax.fori_loop(wid * units_per_worker, (wid + 1) * units_per_worker,
                      unit_body, 0)

    out_flat = sc_kernel(ids_t, table, gamma, beta)
    out5 = out_flat.reshape(hist, 4, bhi_per_b, 8, 128)
    return out5.transpose(2, 4, 0, 1, 3).reshape(bsz, hist, EMB)


# speedup vs baseline: 1.2566x; 1.0849x over previous
import functools

import jax
import jax.numpy as jnp
from jax import lax
from jax.experimental import pallas as pl
from jax.experimental.pallas import tpu as pltpu
from jax.experimental.pallas import tpu_sc as plsc

EMB = 32
EPS = 1e-12
HALF = 16
NW = 32
CHUNK = 128
UNIT_B = 1024
STG = 4 * 8 * UNIT_B


def kernel(entity_ids, table, gamma, beta):
    bsz, hist = entity_ids.shape
    nrows = bsz * hist
    rows_pw = nrows // NW
    nchunks = rows_pw // CHUNK
    units_pw = rows_pw // UNIT_B
    cpu_ = UNIT_B // CHUNK
    bhi_per_b = bsz // 128
    h_stride = EMB * bsz
    ids_flat = entity_ids.astype(jnp.int32).T.reshape(nrows)

    mesh = plsc.VectorSubcoreMesh(core_axis_name="c", subcore_axis_name="s")

    @functools.partial(
        pl.kernel,
        out_type=jax.ShapeDtypeStruct((nrows * EMB,), jnp.float32),
        mesh=mesh,
        scratch_types=[
            pltpu.VMEM((rows_pw,), jnp.int32),
            pltpu.VMEM((4, CHUNK, EMB), jnp.float32),
            pltpu.VMEM((2 * STG,), jnp.float32),
            pltpu.VMEM((EMB,), jnp.float32),
            pltpu.VMEM((EMB,), jnp.float32),
            pltpu.SemaphoreType.DMA,
            pltpu.SemaphoreType.DMA,
            pltpu.SemaphoreType.DMA,
            pltpu.SemaphoreType.DMA,
            pltpu.SemaphoreType.DMA((2,)),
        ],
        compiler_params=pltpu.CompilerParams(
            needs_layout_passes=False, use_tc_tiling_on_sc=False),
    )
    def sc_kernel(ids_hbm, table_hbm, gamma_hbm, beta_hbm, out_hbm,
                  idx_v, data_v, stage_v, gam_v, bet_v,
                  gsem0, gsem1, gsem2, gsem3, ssem):
        gsem = (gsem0, gsem1, gsem2, gsem3)
        wid = lax.axis_index("s") * 2 + lax.axis_index("c")
        pltpu.sync_copy(ids_hbm.at[pl.ds(wid * rows_pw, rows_pw)], idx_v)
        pltpu.sync_copy(gamma_hbm, gam_v)
        pltpu.sync_copy(beta_hbm, bet_v)
        g0 = gam_v[pl.ds(0, HALF)]
        g1 = gam_v[pl.ds(HALF, HALF)]
        b0 = bet_v[pl.ds(0, HALF)]
        b1 = bet_v[pl.ds(HALF, HALF)]
        lane15 = jnp.full((HALF, 1), 15, jnp.int32)
        dnums = lax.GatherDimensionNumbers(
            offset_dims=(), collapsed_slice_dims=(0,), start_index_map=(0,))

        def bcast_last(x):
            return lax.gather(x, lane15, dnums, (1,),
                              mode=lax.GatherScatterMode.PROMISE_IN_BOUNDS)

        iota = lax.iota(jnp.int32, HALF)
        k01 = (iota // 8) * (8 * UNIT_B) + (iota % 8) * 128
        k23 = k01 + 2 * (8 * UNIT_B)
        u0 = wid * units_pw

        def start_gather(k, slot):
            pltpu.async_copy(
                table_hbm.at[idx_v.at[pl.ds(k * CHUNK, CHUNK)]],
                data_v.at[slot], gsem[slot])

        def stores(su, out_off, wait):
            for ch in range(4):
                cp = pltpu.make_async_copy(
                    stage_v.at[pl.ds(su * STG + ch * 8 * UNIT_B, 8 * UNIT_B)],
                    out_hbm.at[pl.ds(out_off + ch * (8 * bsz), 8 * UNIT_B)],
                    ssem.at[su])
                if wait:
                    cp.wait()
                else:
                    cp.start()

        def process_chunk(k, slot):
            pltpu.make_async_copy(
                table_hbm.at[idx_v.at[pl.ds(k * CHUNK, CHUNK)]],
                data_v.at[slot], gsem[slot]).wait()
            base = ((k // cpu_) % 2) * STG + (k % cpu_) * UNIT_B

            def row(r):
                v0 = data_v[slot, r, pl.ds(0, HALF)]
                v1 = data_v[slot, r, pl.ds(HALF, HALF)]
                s = bcast_last(jnp.cumsum(v0 + v1))
                q2 = bcast_last(jnp.cumsum(v0 * v0 + v1 * v1))
                mean = s * (1.0 / EMB)
                var = jnp.maximum(q2 * (1.0 / EMB) - mean * mean, 0.0) + EPS
                i = lax.bitcast_convert_type(var, jnp.int32)
                i = jnp.int32(0x5F3759DF) - lax.shift_right_logical(i, 1)
                y = lax.bitcast_convert_type(i, jnp.float32)
                xh = var * 0.5
                y = y * (1.5 - xh * y * y)
                y = y * (1.5 - xh * y * y)
                y = y * (1.5 - xh * y * y)
                pos = base + r
                plsc.store_scatter(stage_v, [k01 + pos],
                                   (v0 - mean) * (y * g0) + b0)
                plsc.store_scatter(stage_v, [k23 + pos],
                                   (v1 - mean) * (y * g1) + b1)

            plsc.parallel_loop(0, CHUNK, 1, unroll=8)(row)

        def flush(k):
            m = k // cpu_
            u = u0 + m
            out_off = (u // 4) * h_stride + (u % 4) * (8 * UNIT_B)
            stores(m % 2, out_off, wait=False)

            @pl.when(m >= 1)
            def _():
                stores((m - 1) % 2, 0, wait=True)

        for k in range(3):
            start_gather(k, k)

        def quad(p, _):
            for j in range(4):
                k = 4 * p + j
                process_chunk(k, j)

                @pl.when(k + 3 < nchunks)
                def _():
                    start_gather(k + 3, (j + 3) % 4)

            @pl.when(p % 2 == 1)
            def _():
                flush(4 * p + 3)
            return 0

        lax.fori_loop(0, nchunks // 4, quad, 0)
        stores((units_pw - 1) % 2, 0, wait=True)

    out_flat = sc_kernel(ids_flat, table, gamma, beta)
    out5 = out_flat.reshape(hist, 4, bhi_per_b, 8, 128)
    return out5.transpose(2, 4, 0, 1, 3).reshape(bsz, hist, EMB)


# speedup vs baseline: 1.4983x; 1.1924x over previous
import functools

import jax
import jax.numpy as jnp
from jax import lax
from jax.experimental import pallas as pl
from jax.experimental.pallas import tpu as pltpu
from jax.experimental.pallas import tpu_sc as plsc

EMB = 32
EPS = 1e-12
HALF = 16
NW = 32
CHUNK = 128
UNIT_B = 1024
PITCH = 129


def kernel(entity_ids, table, gamma, beta):
    bsz, hist = entity_ids.shape
    nrows = bsz * hist
    rows_pw = nrows // NW
    nchunks = rows_pw // CHUNK
    units_pw = rows_pw // UNIT_B
    cpu_ = UNIT_B // CHUNK
    bhi_per_b = bsz // 128
    h_stride = EMB * bsz
    ids_flat = entity_ids.astype(jnp.int32).T.reshape(nrows)

    mesh = plsc.VectorSubcoreMesh(core_axis_name="c", subcore_axis_name="s")

    @functools.partial(
        pl.kernel,
        out_type=jax.ShapeDtypeStruct((hist, 4, bhi_per_b, 8, 128),
                                      jnp.float32),
        mesh=mesh,
        scratch_types=[
            pltpu.VMEM((rows_pw,), jnp.int32),
            pltpu.VMEM((4, CHUNK, EMB), jnp.float32),
            pltpu.VMEM((2, 4, 8, 8, PITCH), jnp.float32),
            pltpu.VMEM((EMB,), jnp.float32),
            pltpu.VMEM((EMB,), jnp.float32),
            pltpu.SemaphoreType.DMA,
            pltpu.SemaphoreType.DMA,
            pltpu.SemaphoreType.DMA,
            pltpu.SemaphoreType.DMA,
            pltpu.SemaphoreType.DMA((2,)),
        ],
        compiler_params=pltpu.CompilerParams(
            needs_layout_passes=False, use_tc_tiling_on_sc=False),
    )
    def sc_kernel(ids_hbm, table_hbm, gamma_hbm, beta_hbm, out_hbm,
                  idx_v, data_v, stage_v, gam_v, bet_v,
                  gsem0, gsem1, gsem2, gsem3, ssem):
        gsem = (gsem0, gsem1, gsem2, gsem3)
        wid = lax.axis_index("s") * 2 + lax.axis_index("c")
        pltpu.sync_copy(ids_hbm.at[pl.ds(wid * rows_pw, rows_pw)], idx_v)
        pltpu.sync_copy(gamma_hbm, gam_v)
        pltpu.sync_copy(beta_hbm, bet_v)
        g0 = gam_v[pl.ds(0, HALF)]
        g1 = gam_v[pl.ds(HALF, HALF)]
        b0 = bet_v[pl.ds(0, HALF)]
        b1 = bet_v[pl.ds(HALF, HALF)]
        lane15 = jnp.full((HALF, 1), 15, jnp.int32)
        dnums = lax.GatherDimensionNumbers(
            offset_dims=(), collapsed_slice_dims=(0,), start_index_map=(0,))

        def bcast_last(x):
            return lax.gather(x, lane15, dnums, (1,),
                              mode=lax.GatherScatterMode.PROMISE_IN_BOUNDS)

        iota = lax.iota(jnp.int32, HALF)
        d_ch = iota // 8
        d_clo = iota % 8
        u0 = wid * units_pw

        def start_gather(k, slot):
            pltpu.async_copy(
                table_hbm.at[idx_v.at[pl.ds(k * CHUNK, CHUNK)]],
                data_v.at[slot], gsem[slot])

        def stores(su, h, q8, wait):
            for ch in range(4):
                cp = pltpu.make_async_copy(
                    stage_v.at[su, ch, :, :, pl.ds(0, 128)],
                    out_hbm.at[h, ch, pl.ds(q8, 8)],
                    ssem.at[su])
                if wait:
                    cp.wait()
                else:
                    cp.start()

        def process_chunk(k, slot):
            pltpu.make_async_copy(
                table_hbm.at[idx_v.at[pl.ds(k * CHUNK, CHUNK)]],
                data_v.at[slot], gsem[slot]).wait()
            d_su = jnp.full((HALF,), (k // cpu_) % 2, jnp.int32)
            d_bhi = jnp.full((HALF,), k % cpu_, jnp.int32)

            def row(r):
                v0 = data_v[slot, r, pl.ds(0, HALF)]
                v1 = data_v[slot, r, pl.ds(HALF, HALF)]
                s = bcast_last(jnp.cumsum(v0 + v1))
                q2 = bcast_last(jnp.cumsum(v0 * v0 + v1 * v1))
                mean = s * (1.0 / EMB)
                var = jnp.maximum(q2 * (1.0 / EMB) - mean * mean, 0.0) + EPS
                i = lax.bitcast_convert_type(var, jnp.int32)
                i = jnp.int32(0x5F3759DF) - lax.shift_right_logical(i, 1)
                y = lax.bitcast_convert_type(i, jnp.float32)
                xh = var * 0.5
                y = y * (1.5 - xh * y * y)
                y = y * (1.5 - xh * y * y)
                y = y * (1.5 - xh * y * y)
                d_r = jnp.full((HALF,), r, jnp.int32)
                plsc.store_scatter(stage_v, [d_su, d_ch, d_bhi, d_clo, d_r],
                                   (v0 - mean) * (y * g0) + b0)
                plsc.store_scatter(stage_v,
                                   [d_su, d_ch + 2, d_bhi, d_clo, d_r],
                                   (v1 - mean) * (y * g1) + b1)

            plsc.parallel_loop(0, CHUNK, 1, unroll=8)(row)

        def flush(k):
            m = k // cpu_
            u = u0 + m
            stores(m % 2, u // 4, (u % 4) * 8, wait=False)

            @pl.when(m >= 1)
            def _():
                stores((m - 1) % 2, 0, 0, wait=True)

        for k in range(3):
            start_gather(k, k)

        def quad(p, _):
            for j in range(4):
                k = 4 * p + j
                process_chunk(k, j)

                @pl.when(k + 3 < nchunks)
                def _():
                    start_gather(k + 3, (j + 3) % 4)

            @pl.when(p % 2 == 1)
            def _():
                flush(4 * p + 3)
            return 0

        lax.fori_loop(0, nchunks // 4, quad, 0)
        stores((units_pw - 1) % 2, 0, 0, wait=True)

    out5 = sc_kernel(ids_flat, table, gamma, beta)
    return out5.transpose(2, 4, 0, 1, 3).reshape(bsz, hist, EMB)


# speedup vs baseline: 1.6221x; 1.0826x over previous
import functools

import jax
import jax.numpy as jnp
from jax import lax
from jax.experimental import pallas as pl
from jax.experimental.pallas import tpu as pltpu
from jax.experimental.pallas import tpu_sc as plsc

EMB = 32
EPS = 1e-12
HALF = 16
NW = 32
CHUNK = 128
UNIT_B = 1024
PITCH = 129


def kernel(entity_ids, table, gamma, beta):
    bsz, hist = entity_ids.shape
    nrows = bsz * hist
    rows_pw = nrows // NW
    nchunks = rows_pw // CHUNK
    units_pw = rows_pw // UNIT_B
    cpu_ = UNIT_B // CHUNK
    bhi_per_b = bsz // 128
    h_stride = EMB * bsz
    ids_flat = entity_ids.astype(jnp.int32).T.reshape(nrows)

    mesh = plsc.VectorSubcoreMesh(core_axis_name="c", subcore_axis_name="s")

    @functools.partial(
        pl.kernel,
        out_type=jax.ShapeDtypeStruct((hist, 4, bhi_per_b, 8, 128),
                                      jnp.float32),
        mesh=mesh,
        scratch_types=[
            pltpu.VMEM((rows_pw,), jnp.int32),
            pltpu.VMEM((4, CHUNK, EMB), jnp.float32),
            pltpu.VMEM((2, 4, 8, 8, PITCH), jnp.float32),
            pltpu.VMEM((EMB,), jnp.float32),
            pltpu.VMEM((EMB,), jnp.float32),
            pltpu.SemaphoreType.DMA,
            pltpu.SemaphoreType.DMA,
            pltpu.SemaphoreType.DMA,
            pltpu.SemaphoreType.DMA,
            pltpu.SemaphoreType.DMA((2,)),
        ],
        compiler_params=pltpu.CompilerParams(
            needs_layout_passes=False, use_tc_tiling_on_sc=False),
    )
    def sc_kernel(ids_hbm, table_hbm, gamma_hbm, beta_hbm, out_hbm,
                  idx_v, data_v, stage_v, gam_v, bet_v,
                  gsem0, gsem1, gsem2, gsem3, ssem):
        gsem = (gsem0, gsem1, gsem2, gsem3)
        wid = lax.axis_index("s") * 2 + lax.axis_index("c")
        pltpu.sync_copy(ids_hbm.at[pl.ds(wid * rows_pw, rows_pw)], idx_v)
        pltpu.sync_copy(gamma_hbm, gam_v)
        pltpu.sync_copy(beta_hbm, bet_v)
        g0 = gam_v[pl.ds(0, HALF)]
        g1 = gam_v[pl.ds(HALF, HALF)]
        b0 = bet_v[pl.ds(0, HALF)]
        b1 = bet_v[pl.ds(HALF, HALF)]
        lane15 = jnp.full((HALF, 1), 15, jnp.int32)
        dnums = lax.GatherDimensionNumbers(
            offset_dims=(), collapsed_slice_dims=(0,), start_index_map=(0,))

        def bcast_last(x):
            return lax.gather(x, lane15, dnums, (1,),
                              mode=lax.GatherScatterMode.PROMISE_IN_BOUNDS)

        iota = lax.iota(jnp.int32, HALF)
        d_ch = iota // 8
        d_clo = iota % 8
        u0 = wid * units_pw

        def start_gather(k, slot):
            pltpu.async_copy(
                table_hbm.at[idx_v.at[pl.ds(k * CHUNK, CHUNK)]],
                data_v.at[slot], gsem[slot])

        def stores(su, h, q8, wait):
            for ch in range(4):
                cp = pltpu.make_async_copy(
                    stage_v.at[su, ch, :, :, pl.ds(0, 128)],
                    out_hbm.at[h, ch, pl.ds(q8, 8)],
                    ssem.at[su])
                if wait:
                    cp.wait()
                else:
                    cp.start()

        def process_chunk(k, slot):
            pltpu.make_async_copy(
                table_hbm.at[idx_v.at[pl.ds(k * CHUNK, CHUNK)]],
                data_v.at[slot], gsem[slot]).wait()
            d_su = jnp.full((HALF,), (k // cpu_) % 2, jnp.int32)
            d_bhi = jnp.full((HALF,), k % cpu_, jnp.int32)

            def row(r):
                v0 = data_v[slot, r, pl.ds(0, HALF)]
                v1 = data_v[slot, r, pl.ds(HALF, HALF)]
                s = bcast_last(jnp.cumsum(v0 + v1))
                q2 = bcast_last(jnp.cumsum(v0 * v0 + v1 * v1))
                mean = s * (1.0 / EMB)
                var = jnp.maximum(q2 * (1.0 / EMB) - mean * mean, 0.0) + EPS
                i = lax.bitcast_convert_type(var, jnp.int32)
                i = jnp.int32(0x5F3759DF) - lax.shift_right_logical(i, 1)
                y = lax.bitcast_convert_type(i, jnp.float32)
                xh = var * 0.5
                y = y * (1.5 - xh * y * y)
                y = y * (1.5 - xh * y * y)
                d_r = jnp.full((HALF,), r, jnp.int32)
                plsc.store_scatter(stage_v, [d_su, d_ch, d_bhi, d_clo, d_r],
                                   (v0 - mean) * (y * g0) + b0)
                plsc.store_scatter(stage_v,
                                   [d_su, d_ch + 2, d_bhi, d_clo, d_r],
                                   (v1 - mean) * (y * g1) + b1)

            plsc.parallel_loop(0, CHUNK, 1, unroll=8)(row)

        def flush(k):
            m = k // cpu_
            u = u0 + m
            stores(m % 2, u // 4, (u % 4) * 8, wait=False)

            @pl.when(m >= 1)
            def _():
                stores((m - 1) % 2, 0, 0, wait=True)

        for k in range(3):
            start_gather(k, k)

        def quad(p, _):
            for j in range(4):
                k = 4 * p + j
                process_chunk(k, j)

                @pl.when(k + 3 < nchunks)
                def _():
                    start_gather(k + 3, (j + 3) % 4)

            @pl.when(p % 2 == 1)
            def _():
                flush(4 * p + 3)
            return 0

        lax.fori_loop(0, nchunks // 4, quad, 0)
        stores((units_pw - 1) % 2, 0, 0, wait=True)

    out5 = sc_kernel(ids_flat, table, gamma, beta)
    return out5.transpose(2, 4, 0, 1, 3).reshape(bsz, hist, EMB)
